# baseline (speedup 1.0000x reference)
import functools

import jax
import jax.numpy as jnp
from jax import lax
from jax.experimental import pallas as pl
from jax.experimental.pallas import tpu as pltpu
from jax.experimental.pallas import tpu_sc as plsc

NC = 2
NS = 16
NW = NC * NS
L = 16
K = 48
SHIFT = 14
MASK = (1 << SHIFT) - 1
CORE0_FRAC = 0.52
RING = 4
EPS = 1e-12


def _sc_spmm(h, packed1, vals1, chunks0, chunks1):
    n, d = h.shape
    q0 = chunks0 * K
    q1 = chunks1 * K
    n_pad = 10240 if n == 10000 else ((n + NS * 8 - 1) // (NS * 8)) * NS * 8
    rows_per_tile = n_pad // NS
    wb = 40
    n_wb = rows_per_tile // wb
    mesh = plsc.VectorSubcoreMesh(core_axis_name="c", subcore_axis_name="s")

    @functools.partial(
        pl.kernel,
        mesh=mesh,
        out_type=jax.ShapeDtypeStruct((NC, n_pad, d), jnp.float32),
        scratch_types=[
            pltpu.VMEM_SHARED((n_pad, d), jnp.float32),
            pltpu.VMEM((q0,), jnp.int32),
            pltpu.VMEM((q0,), jnp.float32),
            pltpu.VMEM((RING, K), jnp.int32),
            pltpu.VMEM((RING, K), jnp.int32),
            pltpu.VMEM((K, d), jnp.float32),
            pltpu.VMEM((K, d), jnp.float32),
            pltpu.VMEM((K, d), jnp.float32),
            pltpu.VMEM((K, d), jnp.float32),
            pltpu.SemaphoreType.DMA,
            pltpu.SemaphoreType.DMA,
            pltpu.SemaphoreType.DMA,
            pltpu.SemaphoreType.DMA,
            pltpu.SemaphoreType.DMA,
            pltpu.SemaphoreType.DMA,
            pltpu.SemaphoreType.DMA,
            pltpu.SemaphoreType.DMA,
        ],
    )
    def spmm(h_hbm, packed_hbm, vals_hbm, out_hbm,
             accum, packed_v, vals_v, srcring, dstring,
             buf0, buf1, buf2, buf3,
             gsem0, gsem1, gsem2, gsem3, ssem0, ssem1, ssem2, ssem3):
        cid = lax.axis_index("c")
        sid = lax.axis_index("s")
        zero = jnp.zeros((L,), jnp.float32)
        bufs = (buf0, buf1, buf2, buf3)
        gsems = (gsem0, gsem1, gsem2, gsem3)
        ssems = (ssem0, ssem1, ssem2, ssem3)
        my_chunks = jnp.where(cid == 0, chunks0, chunks1)

        def zrow(r, carry):
            for j in range(d // L):
                buf0[r, pl.ds(j * L, L)] = zero
            return carry
        lax.fori_loop(0, min(K, wb), zrow, 0)
        base = sid * rows_per_tile
        for i in range(n_wb):
            pltpu.sync_copy(buf0.at[pl.ds(0, wb)],
                            accum.at[pl.ds(base + i * wb, wb)])
        plsc.subcore_barrier()

        ebase = jnp.where(cid == 0, sid * q0, NS * q0 + sid * q1)
        pltpu.sync_copy(packed_hbm.at[pl.ds(ebase, q0)], packed_v)
        pltpu.sync_copy(vals_hbm.at[pl.ds(ebase, q0)], vals_v)

        def unpack_src(ci, p):
            for g in range(K // L):
                pv = packed_v[pl.ds(ci * K + g * L, L)]
                srcring[p, pl.ds(g * L, L)] = pv & MASK

        def gather(ci, p):
            return pltpu.async_copy(
                h_hbm.at[srcring.at[p]], bufs[p], gsems[p])

        def scatter(ci, p):
            for g in range(K // L):
                pv = packed_v[pl.ds(ci * K + g * L, L)]
                dvec = lax.shift_right_logical(pv, SHIFT)
                pltpu.async_copy(bufs[p].at[pl.ds(g * L, L)],
                                 accum.at[dvec], ssems[p], add=True)

        def scatter_wait(p):
            zvec = jnp.zeros((L,), jnp.int32)
            for g in range(K // L):
                pltpu.make_async_copy(bufs[p].at[pl.ds(g * L, L)],
                                      accum.at[zvec], ssems[p]).wait()

        for c in range(RING - 1):
            unpack_src(c, c)
            gather(c, c)

        def step(ci, p):
            pltpu.make_async_copy(
                h_hbm.at[srcring.at[p]], bufs[p], gsems[p]).wait()

            buf = bufs[p]

            def scale(g, c2):
                vvec = vals_v[pl.ds(ci * K + g * L, L)]
                for e16 in range(L):
                    vv = jnp.full((L,), vvec[e16], jnp.float32)
                    r = g * L + e16
                    for j in range(d // L):
                        sl = pl.ds(j * L, L)
                        buf[r, sl] = buf[r, sl] * vv
                return c2
            lax.fori_loop(0, K // L, scale, 0)

            scatter(ci, p)

            pnext = (p + 3) % RING

            @pl.when(ci > 0)
            def _():
                scatter_wait(pnext)

            @pl.when(ci + 3 < my_chunks)
            def _():
                unpack_src(ci + 3, pnext)
                gather(ci + 3, pnext)

        def quad(ti, carry):
            c0 = ti * RING
            for u in range(RING):
                step(c0 + u, u)
            return carry
        lax.fori_loop(0, my_chunks // RING, quad, 0)
        scatter_wait(RING - 1)

        plsc.subcore_barrier()
        for i in range(n_wb):
            r0 = base + i * wb
            pltpu.sync_copy(accum.at[pl.ds(r0, wb)], buf0.at[pl.ds(0, wb)])
            pltpu.sync_copy(buf0.at[pl.ds(0, wb)],
                            out_hbm.at[cid, pl.ds(r0, wb)])

    return spmm(h, packed1, vals1)


def _tc_combine(parts, n):
    _, _, d = parts.shape
    blk = 2000

    def body(p_ref, o_ref):
        o_ref[...] = p_ref[0] + p_ref[1]

    return pl.pallas_call(
        body,
        grid=(n // blk,),
        in_specs=[pl.BlockSpec((2, blk, d), lambda i: (0, i, 0))],
        out_specs=pl.BlockSpec((blk, d), lambda i: (i, 0)),
        out_shape=jax.ShapeDtypeStruct((n, d), jnp.float32),
    )(parts)


def _tc_fuse(parts2, h1, t):
    n, d = h1.shape
    blk = 2000

    def body(p_ref, h1_ref, t_ref, o_ref):
        h2 = p_ref[0] + p_ref[1]
        lay = (h1_ref[...] + h2) * 0.5
        nrm = jnp.sqrt(jnp.sum(lay * lay, axis=-1, keepdims=True))
        o_ref[...] = lay / jnp.maximum(nrm, EPS) + t_ref[...]

    return pl.pallas_call(
        body,
        grid=(n // blk,),
        in_specs=[
            pl.BlockSpec((2, blk, d), lambda i: (0, i, 0)),
            pl.BlockSpec((blk, d), lambda i: (i, 0)),
            pl.BlockSpec((blk, d), lambda i: (i, 0)),
        ],
        out_specs=pl.BlockSpec((blk, d), lambda i: (i, 0)),
        out_shape=jax.ShapeDtypeStruct((n, d), jnp.float32),
    )(parts2, h1, t)


def kernel(x, edge_index, edge_vals):
    n, d = x.shape
    e = edge_vals.shape[0]
    dst = edge_index[0]
    src = edge_index[1]

    total_chunks = (e + NS * K - 1) // (NS * K)
    chunks0 = max(RING, int(round(total_chunks * CORE0_FRAC / RING)) * RING)
    q0 = chunks0 * K
    rem = e - NS * q0
    chunks1 = max(RING, -(-rem // (NS * K * RING)) * RING) if rem > 0 else RING
    q1 = chunks1 * K
    e_pad = NS * (q0 + q1) + max(q0 - q1, 0)
    pad = e_pad - e
    packed1 = jnp.pad(src, (0, pad)) | (jnp.pad(dst, (0, pad)) << SHIFT)
    vals1 = jnp.pad(edge_vals, (0, pad))

    t = x
    for _ in range(3):
        p1 = _sc_spmm(t, packed1, vals1, chunks0, chunks1)
        h1 = _tc_combine(p1, n)
        p2 = _sc_spmm(h1, packed1, vals1, chunks0, chunks1)
        t = _tc_fuse(p2, h1, t)
    return t


# speedup vs baseline: 1.4639x; 1.4639x over previous
import functools

import jax
import jax.numpy as jnp
from jax import lax
from jax.experimental import pallas as pl
from jax.experimental.pallas import tpu as pltpu
from jax.experimental.pallas import tpu_sc as plsc

NC = 2
NS = 16
NW = NC * NS
L = 16
K = 48
SHIFT = 14
MASK = (1 << SHIFT) - 1
CORE0_FRAC = 0.52
RING = 3
EPS = 1e-12


def _sc_spmm(h, packed1, vals1, chunks0, chunks1):
    n, d = h.shape
    q0 = chunks0 * K
    q1 = chunks1 * K
    n_pad = 10240 if n == 10000 else ((n + NS * 8 - 1) // (NS * 8)) * NS * 8
    rows_per_tile = n_pad // NS
    wb = 40
    n_wb = rows_per_tile // wb
    mesh = plsc.VectorSubcoreMesh(core_axis_name="c", subcore_axis_name="s")

    @functools.partial(
        pl.kernel,
        mesh=mesh,
        out_type=jax.ShapeDtypeStruct((NC, n_pad, d), jnp.float32),
        scratch_types=[
            pltpu.VMEM_SHARED((n_pad, d), jnp.float32),
            pltpu.VMEM((q0,), jnp.int32),
            pltpu.VMEM((q0,), jnp.float32),
            pltpu.VMEM((RING, K), jnp.int32),
        ] + [pltpu.VMEM((K, d), jnp.float32) for _ in range(RING)]
          + [pltpu.SemaphoreType.DMA for _ in range(2 * RING)],
    )
    def spmm(h_hbm, packed_hbm, vals_hbm, out_hbm,
             accum, packed_v, vals_v, srcring, *ring_scratch):
        cid = lax.axis_index("c")
        sid = lax.axis_index("s")
        zero = jnp.zeros((L,), jnp.float32)
        bufs = ring_scratch[:RING]
        gsems = ring_scratch[RING:2 * RING]
        ssems = ring_scratch[2 * RING:]
        buf0 = bufs[0]
        my_chunks = jnp.where(cid == 0, chunks0, chunks1)

        def zrow(r, carry):
            for j in range(d // L):
                buf0[r, pl.ds(j * L, L)] = zero
            return carry
        lax.fori_loop(0, min(K, wb), zrow, 0)
        base = sid * rows_per_tile
        for i in range(n_wb):
            pltpu.sync_copy(buf0.at[pl.ds(0, wb)],
                            accum.at[pl.ds(base + i * wb, wb)])
        plsc.subcore_barrier()

        ebase = jnp.where(cid == 0, sid * q0, NS * q0 + sid * q1)
        pltpu.sync_copy(packed_hbm.at[pl.ds(ebase, q0)], packed_v)
        pltpu.sync_copy(vals_hbm.at[pl.ds(ebase, q0)], vals_v)

        def unpack_src(ci, p):
            for g in range(K // L):
                pv = packed_v[pl.ds(ci * K + g * L, L)]
                srcring[p, pl.ds(g * L, L)] = pv & MASK

        def gather(ci, p):
            return pltpu.async_copy(
                h_hbm.at[srcring.at[p]], bufs[p], gsems[p])

        def scatter(ci, p):
            for g in range(K // L):
                pv = packed_v[pl.ds(ci * K + g * L, L)]
                dvec = lax.shift_right_logical(pv, SHIFT)
                pltpu.async_copy(bufs[p].at[pl.ds(g * L, L)],
                                 accum.at[dvec], ssems[p], add=True)

        def scatter_wait(p):
            zvec = jnp.zeros((L,), jnp.int32)
            for g in range(K // L):
                pltpu.make_async_copy(bufs[p].at[pl.ds(g * L, L)],
                                      accum.at[zvec], ssems[p]).wait()

        for c in range(RING - 1):
            unpack_src(c, c)
            gather(c, c)

        def step(ci, p):
            pltpu.make_async_copy(
                h_hbm.at[srcring.at[p]], bufs[p], gsems[p]).wait()

            buf = bufs[p]

            def scale(g, c2):
                vvec = vals_v[pl.ds(ci * K + g * L, L)]
                for e16 in range(L):
                    vv = jnp.full((L,), vvec[e16], jnp.float32)
                    r = g * L + e16
                    for j in range(d // L):
                        sl = pl.ds(j * L, L)
                        buf[r, sl] = buf[r, sl] * vv
                return c2
            lax.fori_loop(0, K // L, scale, 0)

            scatter(ci, p)

            pnext = (p + RING - 1) % RING

            @pl.when(ci > 0)
            def _():
                scatter_wait(pnext)

            @pl.when(ci + RING - 1 < my_chunks)
            def _():
                unpack_src(ci + RING - 1, pnext)
                gather(ci + RING - 1, pnext)

        def quad(ti, carry):
            c0 = ti * RING
            for u in range(RING):
                step(c0 + u, u)
            return carry
        lax.fori_loop(0, my_chunks // RING, quad, 0)
        scatter_wait(RING - 1)

        plsc.subcore_barrier()
        for i in range(n_wb):
            r0 = base + i * wb
            pltpu.sync_copy(accum.at[pl.ds(r0, wb)], buf0.at[pl.ds(0, wb)])
            pltpu.sync_copy(buf0.at[pl.ds(0, wb)],
                            out_hbm.at[cid, pl.ds(r0, wb)])

    return spmm(h, packed1, vals1)


def _tc_combine(parts, n):
    _, _, d = parts.shape
    blk = 2000

    def body(p_ref, o_ref):
        o_ref[...] = p_ref[0] + p_ref[1]

    return pl.pallas_call(
        body,
        grid=(n // blk,),
        in_specs=[pl.BlockSpec((2, blk, d), lambda i: (0, i, 0))],
        out_specs=pl.BlockSpec((blk, d), lambda i: (i, 0)),
        out_shape=jax.ShapeDtypeStruct((n, d), jnp.float32),
    )(parts)


def _tc_fuse(parts2, h1, t):
    n, d = h1.shape
    blk = 2000

    def body(p_ref, h1_ref, t_ref, o_ref):
        h2 = p_ref[0] + p_ref[1]
        lay = (h1_ref[...] + h2) * 0.5
        nrm = jnp.sqrt(jnp.sum(lay * lay, axis=-1, keepdims=True))
        o_ref[...] = lay / jnp.maximum(nrm, EPS) + t_ref[...]

    return pl.pallas_call(
        body,
        grid=(n // blk,),
        in_specs=[
            pl.BlockSpec((2, blk, d), lambda i: (0, i, 0)),
            pl.BlockSpec((blk, d), lambda i: (i, 0)),
            pl.BlockSpec((blk, d), lambda i: (i, 0)),
        ],
        out_specs=pl.BlockSpec((blk, d), lambda i: (i, 0)),
        out_shape=jax.ShapeDtypeStruct((n, d), jnp.float32),
    )(parts2, h1, t)


def kernel(x, edge_index, edge_vals):
    n, d = x.shape
    e = edge_vals.shape[0]
    dst = edge_index[0]
    src = edge_index[1]

    total_chunks = (e + NS * K - 1) // (NS * K)
    chunks0 = max(RING, int(round(total_chunks * CORE0_FRAC / RING)) * RING)
    q0 = chunks0 * K
    rem = e - NS * q0
    chunks1 = max(RING, -(-rem // (NS * K * RING)) * RING) if rem > 0 else RING
    q1 = chunks1 * K
    e_pad = NS * (q0 + q1) + max(q0 - q1, 0)
    pad = e_pad - e
    packed1 = jnp.pad(src, (0, pad)) | (jnp.pad(dst, (0, pad)) << SHIFT)
    vals1 = jnp.pad(edge_vals, (0, pad))

    t = x
    for _ in range(3):
        p1 = _sc_spmm(t, packed1, vals1, chunks0, chunks1)
        h1 = _tc_combine(p1, n)
        p2 = _sc_spmm(h1, packed1, vals1, chunks0, chunks1)
        t = _tc_fuse(p2, h1, t)
    return t


# speedup vs baseline: 1.5634x; 1.0680x over previous
import functools

import jax
import jax.numpy as jnp
from jax import lax
from jax.experimental import pallas as pl
from jax.experimental.pallas import tpu as pltpu
from jax.experimental.pallas import tpu_sc as plsc

NC = 2
NS = 16
NW = NC * NS
L = 16
K = 48
SHIFT = 14
MASK = (1 << SHIFT) - 1
CORE0_FRAC = 0.55
RING0 = 4
RING1 = 3
RING = RING0
EPS = 1e-12


def _sc_spmm(h, packed1, vals1, chunks0, chunks1):
    n, d = h.shape
    q0 = chunks0 * K
    q1 = chunks1 * K
    n_pad = 10240 if n == 10000 else ((n + NS * 8 - 1) // (NS * 8)) * NS * 8
    rows_per_tile = n_pad // NS
    wb = 40
    n_wb = rows_per_tile // wb
    mesh = plsc.VectorSubcoreMesh(core_axis_name="c", subcore_axis_name="s")

    @functools.partial(
        pl.kernel,
        mesh=mesh,
        out_type=jax.ShapeDtypeStruct((NC, n_pad, d), jnp.float32),
        scratch_types=[
            pltpu.VMEM_SHARED((n_pad, d), jnp.float32),
            pltpu.VMEM((q0,), jnp.int32),
            pltpu.VMEM((q0,), jnp.float32),
            pltpu.VMEM((RING, K), jnp.int32),
        ] + [pltpu.VMEM((K, d), jnp.float32) for _ in range(RING)]
          + [pltpu.SemaphoreType.DMA for _ in range(2 * RING)],
    )
    def spmm(h_hbm, packed_hbm, vals_hbm, out_hbm,
             accum, packed_v, vals_v, srcring, *ring_scratch):
        cid = lax.axis_index("c")
        sid = lax.axis_index("s")
        zero = jnp.zeros((L,), jnp.float32)
        bufs = ring_scratch[:RING]
        gsems = ring_scratch[RING:2 * RING]
        ssems = ring_scratch[2 * RING:]
        buf0 = bufs[0]

        def zrow(r, carry):
            for j in range(d // L):
                buf0[r, pl.ds(j * L, L)] = zero
            return carry
        lax.fori_loop(0, min(K, wb), zrow, 0)
        base = sid * rows_per_tile
        for i in range(n_wb):
            pltpu.sync_copy(buf0.at[pl.ds(0, wb)],
                            accum.at[pl.ds(base + i * wb, wb)])
        plsc.subcore_barrier()

        ebase = jnp.where(cid == 0, sid * q0, NS * q0 + sid * q1)
        pltpu.sync_copy(packed_hbm.at[pl.ds(ebase, q0)], packed_v)
        pltpu.sync_copy(vals_hbm.at[pl.ds(ebase, q0)], vals_v)

        def unpack_src(ci, p):
            for g in range(K // L):
                pv = packed_v[pl.ds(ci * K + g * L, L)]
                srcring[p, pl.ds(g * L, L)] = pv & MASK

        def gather(ci, p):
            return pltpu.async_copy(
                h_hbm.at[srcring.at[p]], bufs[p], gsems[p])

        def scatter(ci, p):
            for g in range(K // L):
                pv = packed_v[pl.ds(ci * K + g * L, L)]
                dvec = lax.shift_right_logical(pv, SHIFT)
                pltpu.async_copy(bufs[p].at[pl.ds(g * L, L)],
                                 accum.at[dvec], ssems[p], add=True)

        def scatter_wait(p):
            zvec = jnp.zeros((L,), jnp.int32)
            for g in range(K // L):
                pltpu.make_async_copy(bufs[p].at[pl.ds(g * L, L)],
                                      accum.at[zvec], ssems[p]).wait()

        def step(ci, p, ring, nchunks):
            pltpu.make_async_copy(
                h_hbm.at[srcring.at[p]], bufs[p], gsems[p]).wait()

            buf = bufs[p]

            def scale(g, c2):
                vvec = vals_v[pl.ds(ci * K + g * L, L)]
                for e16 in range(L):
                    vv = jnp.full((L,), vvec[e16], jnp.float32)
                    r = g * L + e16
                    for j in range(d // L):
                        sl = pl.ds(j * L, L)
                        buf[r, sl] = buf[r, sl] * vv
                return c2
            lax.fori_loop(0, K // L, scale, 0)

            scatter(ci, p)

            pnext = (p + ring - 1) % ring

            @pl.when(ci > 0)
            def _():
                scatter_wait(pnext)

            @pl.when(ci + ring - 1 < nchunks)
            def _():
                unpack_src(ci + ring - 1, pnext)
                gather(ci + ring - 1, pnext)

        for c in range(RING1 - 1):
            unpack_src(c, c)
            gather(c, c)

        @pl.when(cid == 0)
        def _():
            for c in range(RING1 - 1, RING0 - 1):
                unpack_src(c, c)
                gather(c, c)

            def body0(ti, carry):
                c0 = ti * RING0
                for u in range(RING0):
                    step(c0 + u, u, RING0, chunks0)
                return carry
            lax.fori_loop(0, chunks0 // RING0, body0, 0)
            scatter_wait(RING0 - 1)

        @pl.when(cid != 0)
        def _():
            def body1(ti, carry):
                c0 = ti * RING1
                for u in range(RING1):
                    step(c0 + u, u, RING1, chunks1)
                return carry
            lax.fori_loop(0, chunks1 // RING1, body1, 0)
            scatter_wait(RING1 - 1)

        plsc.subcore_barrier()
        for i in range(n_wb):
            r0 = base + i * wb
            pltpu.sync_copy(accum.at[pl.ds(r0, wb)], buf0.at[pl.ds(0, wb)])
            pltpu.sync_copy(buf0.at[pl.ds(0, wb)],
                            out_hbm.at[cid, pl.ds(r0, wb)])

    return spmm(h, packed1, vals1)


def _tc_combine(parts, n):
    _, _, d = parts.shape
    blk = 2000

    def body(p_ref, o_ref):
        o_ref[...] = p_ref[0] + p_ref[1]

    return pl.pallas_call(
        body,
        grid=(n // blk,),
        in_specs=[pl.BlockSpec((2, blk, d), lambda i: (0, i, 0))],
        out_specs=pl.BlockSpec((blk, d), lambda i: (i, 0)),
        out_shape=jax.ShapeDtypeStruct((n, d), jnp.float32),
    )(parts)


def _tc_fuse(parts2, h1, t):
    n, d = h1.shape
    blk = 2000

    def body(p_ref, h1_ref, t_ref, o_ref):
        h2 = p_ref[0] + p_ref[1]
        lay = (h1_ref[...] + h2) * 0.5
        nrm = jnp.sqrt(jnp.sum(lay * lay, axis=-1, keepdims=True))
        o_ref[...] = lay / jnp.maximum(nrm, EPS) + t_ref[...]

    return pl.pallas_call(
        body,
        grid=(n // blk,),
        in_specs=[
            pl.BlockSpec((2, blk, d), lambda i: (0, i, 0)),
            pl.BlockSpec((blk, d), lambda i: (i, 0)),
            pl.BlockSpec((blk, d), lambda i: (i, 0)),
        ],
        out_specs=pl.BlockSpec((blk, d), lambda i: (i, 0)),
        out_shape=jax.ShapeDtypeStruct((n, d), jnp.float32),
    )(parts2, h1, t)


def kernel(x, edge_index, edge_vals):
    n, d = x.shape
    e = edge_vals.shape[0]
    dst = edge_index[0]
    src = edge_index[1]

    total_chunks = (e + NS * K - 1) // (NS * K)
    chunks0 = max(RING0,
                  int(round(total_chunks * CORE0_FRAC / RING0)) * RING0)
    q0 = chunks0 * K
    rem = e - NS * q0
    chunks1 = (max(RING1, -(-rem // (NS * K * RING1)) * RING1)
               if rem > 0 else RING1)
    q1 = chunks1 * K
    e_pad = NS * (q0 + q1) + max(q0 - q1, 0)
    pad = e_pad - e
    packed1 = jnp.pad(src, (0, pad)) | (jnp.pad(dst, (0, pad)) << SHIFT)
    vals1 = jnp.pad(edge_vals, (0, pad))

    t = x
    for _ in range(3):
        p1 = _sc_spmm(t, packed1, vals1, chunks0, chunks1)
        h1 = _tc_combine(p1, n)
        p2 = _sc_spmm(h1, packed1, vals1, chunks0, chunks1)
        t = _tc_fuse(p2, h1, t)
    return t


# speedup vs baseline: 1.6593x; 1.0613x over previous
import functools

import jax
import jax.numpy as jnp
from jax import lax
from jax.experimental import pallas as pl
from jax.experimental.pallas import tpu as pltpu
from jax.experimental.pallas import tpu_sc as plsc

NC = 2
NS = 16
NW = NC * NS
L = 16
K = 48
SHIFT = 14
MASK = (1 << SHIFT) - 1
CORE0_FRAC = 0.58
RING0 = 4
RING1 = 3
RING = RING0
EPS = 1e-12


def _sc_spmm(h, packed1, vals1, chunks0, chunks1):
    n, d = h.shape
    q0 = chunks0 * K
    q1 = chunks1 * K
    n_pad = 10240 if n == 10000 else ((n + NS * 8 - 1) // (NS * 8)) * NS * 8
    rows_per_tile = n_pad // NS
    wb = 40
    n_wb = rows_per_tile // wb
    mesh = plsc.VectorSubcoreMesh(core_axis_name="c", subcore_axis_name="s")

    @functools.partial(
        pl.kernel,
        mesh=mesh,
        out_type=jax.ShapeDtypeStruct((NC, n_pad, d), jnp.float32),
        scratch_types=[
            pltpu.VMEM_SHARED((n_pad, d), jnp.float32),
            pltpu.VMEM((q0,), jnp.int32),
            pltpu.VMEM((q0,), jnp.float32),
            pltpu.VMEM((RING, K), jnp.int32),
        ] + [pltpu.VMEM((K, d), jnp.float32) for _ in range(RING)]
          + [pltpu.SemaphoreType.DMA for _ in range(2 * RING)],
    )
    def spmm(h_hbm, packed_hbm, vals_hbm, out_hbm,
             accum, packed_v, vals_v, srcring, *ring_scratch):
        cid = lax.axis_index("c")
        sid = lax.axis_index("s")
        zero = jnp.zeros((L,), jnp.float32)
        bufs = ring_scratch[:RING]
        gsems = ring_scratch[RING:2 * RING]
        ssems = ring_scratch[2 * RING:]
        buf0 = bufs[0]

        def zrow(r, carry):
            for j in range(d // L):
                buf0[r, pl.ds(j * L, L)] = zero
            return carry
        lax.fori_loop(0, min(K, wb), zrow, 0)
        base = sid * rows_per_tile
        for i in range(n_wb):
            pltpu.sync_copy(buf0.at[pl.ds(0, wb)],
                            accum.at[pl.ds(base + i * wb, wb)])
        plsc.subcore_barrier()

        ebase = jnp.where(cid == 0, sid * q0, NS * q0 + sid * q1)
        pltpu.sync_copy(packed_hbm.at[pl.ds(ebase, q0)], packed_v)
        pltpu.sync_copy(vals_hbm.at[pl.ds(ebase, q0)], vals_v)

        def unpack_src(ci, p):
            for g in range(K // L):
                pv = packed_v[pl.ds(ci * K + g * L, L)]
                srcring[p, pl.ds(g * L, L)] = pv & MASK

        def gather(ci, p):
            return pltpu.async_copy(
                h_hbm.at[srcring.at[p]], bufs[p], gsems[p])

        def scatter(ci, p):
            for g in range(K // L):
                pv = packed_v[pl.ds(ci * K + g * L, L)]
                dvec = lax.shift_right_logical(pv, SHIFT)
                pltpu.async_copy(bufs[p].at[pl.ds(g * L, L)],
                                 accum.at[dvec], ssems[p], add=True)

        def scatter_wait(p):
            zvec = jnp.zeros((L,), jnp.int32)
            for g in range(K // L):
                pltpu.make_async_copy(bufs[p].at[pl.ds(g * L, L)],
                                      accum.at[zvec], ssems[p]).wait()

        def step(ci, p, ring, nchunks):
            pltpu.make_async_copy(
                h_hbm.at[srcring.at[p]], bufs[p], gsems[p]).wait()

            buf = bufs[p]

            def scale(g, c2):
                vvec = vals_v[pl.ds(ci * K + g * L, L)]
                for e16 in range(L):
                    vv = jnp.full((L,), vvec[e16], jnp.float32)
                    r = g * L + e16
                    for j in range(d // L):
                        sl = pl.ds(j * L, L)
                        buf[r, sl] = buf[r, sl] * vv
                return c2
            lax.fori_loop(0, K // L, scale, 0)

            scatter(ci, p)

            pnext = (p + ring - 1) % ring

            @pl.when(ci > 0)
            def _():
                scatter_wait(pnext)

            @pl.when(ci + ring - 1 < nchunks)
            def _():
                unpack_src(ci + ring - 1, pnext)
                gather(ci + ring - 1, pnext)

        for c in range(RING1 - 1):
            unpack_src(c, c)
            gather(c, c)

        @pl.when(cid == 0)
        def _():
            for c in range(RING1 - 1, RING0 - 1):
                unpack_src(c, c)
                gather(c, c)

            def body0(ti, carry):
                c0 = ti * RING0
                for u in range(RING0):
                    step(c0 + u, u, RING0, chunks0)
                return carry
            lax.fori_loop(0, chunks0 // RING0, body0, 0)
            scatter_wait(RING0 - 1)

        @pl.when(cid != 0)
        def _():
            def body1(ti, carry):
                c0 = ti * RING1
                for u in range(RING1):
                    step(c0 + u, u, RING1, chunks1)
                return carry
            lax.fori_loop(0, chunks1 // RING1, body1, 0)
            scatter_wait(RING1 - 1)

        plsc.subcore_barrier()
        for i in range(n_wb):
            r0 = base + i * wb
            pltpu.sync_copy(accum.at[pl.ds(r0, wb)], buf0.at[pl.ds(0, wb)])
            pltpu.sync_copy(buf0.at[pl.ds(0, wb)],
                            out_hbm.at[cid, pl.ds(r0, wb)])

    return spmm(h, packed1, vals1)


def _tc_combine(parts, n):
    _, _, d = parts.shape
    blk = 2000

    def body(p_ref, o_ref):
        o_ref[...] = p_ref[0] + p_ref[1]

    return pl.pallas_call(
        body,
        grid=(n // blk,),
        in_specs=[pl.BlockSpec((2, blk, d), lambda i: (0, i, 0))],
        out_specs=pl.BlockSpec((blk, d), lambda i: (i, 0)),
        out_shape=jax.ShapeDtypeStruct((n, d), jnp.float32),
    )(parts)


def _tc_fuse(parts2, h1, t):
    n, d = h1.shape
    blk = 2000

    def body(p_ref, h1_ref, t_ref, o_ref):
        h2 = p_ref[0] + p_ref[1]
        lay = (h1_ref[...] + h2) * 0.5
        nrm = jnp.sqrt(jnp.sum(lay * lay, axis=-1, keepdims=True))
        o_ref[...] = lay / jnp.maximum(nrm, EPS) + t_ref[...]

    return pl.pallas_call(
        body,
        grid=(n // blk,),
        in_specs=[
            pl.BlockSpec((2, blk, d), lambda i: (0, i, 0)),
            pl.BlockSpec((blk, d), lambda i: (i, 0)),
            pl.BlockSpec((blk, d), lambda i: (i, 0)),
        ],
        out_specs=pl.BlockSpec((blk, d), lambda i: (i, 0)),
        out_shape=jax.ShapeDtypeStruct((n, d), jnp.float32),
    )(parts2, h1, t)


def kernel(x, edge_index, edge_vals):
    n, d = x.shape
    e = edge_vals.shape[0]
    dst = edge_index[0]
    src = edge_index[1]

    total_chunks = (e + NS * K - 1) // (NS * K)
    chunks0 = max(RING0,
                  int(round(total_chunks * CORE0_FRAC / RING0)) * RING0)
    q0 = chunks0 * K
    rem = e - NS * q0
    chunks1 = (max(RING1, -(-rem // (NS * K * RING1)) * RING1)
               if rem > 0 else RING1)
    q1 = chunks1 * K
    e_pad = NS * (q0 + q1) + max(q0 - q1, 0)
    pad = e_pad - e
    packed1 = jnp.pad(src, (0, pad)) | (jnp.pad(dst, (0, pad)) << SHIFT)
    vals1 = jnp.pad(edge_vals, (0, pad))

    t = x
    for _ in range(3):
        p1 = _sc_spmm(t, packed1, vals1, chunks0, chunks1)
        h1 = _tc_combine(p1, n)
        p2 = _sc_spmm(h1, packed1, vals1, chunks0, chunks1)
        t = _tc_fuse(p2, h1, t)
    return t
